# speedup vs baseline: 1.2017x; 1.2017x over previous
import functools
import jax
import jax.numpy as jnp
from jax.experimental import pallas as pl
from jax.experimental.pallas import tpu as pltpu


def _dense_stage_body(x_ref, wgl_ref, w1_ref, a_ref, h_ref, xw1_ref, hn_ref):
    x = x_ref[...]
    h = jax.lax.dot(x, wgl_ref[...], preferred_element_type=jnp.float32)
    h_ref[...] = h
    xw1_ref[...] = jax.lax.dot(x, w1_ref[...], preferred_element_type=jnp.float32)
    anorm = jnp.sqrt(jnp.sum(a_ref[...] ** 2))
    hn_ref[...] = anorm * jnp.sqrt(jnp.sum(h * h, axis=1, keepdims=True))


def _dense_stage(x, W_gl, W1, a, block_n=1000):
    n, d = x.shape
    grid = n // block_n
    return pl.pallas_call(
        _dense_stage_body,
        grid=(grid,),
        in_specs=[
            pl.BlockSpec((block_n, d), lambda i: (i, 0)),
            pl.BlockSpec((d, W_gl.shape[1]), lambda i: (0, 0)),
            pl.BlockSpec((d, W1.shape[1]), lambda i: (0, 0)),
            pl.BlockSpec((W_gl.shape[1], 1), lambda i: (0, 0)),
        ],
        out_specs=[
            pl.BlockSpec((block_n, W_gl.shape[1]), lambda i: (i, 0)),
            pl.BlockSpec((block_n, W1.shape[1]), lambda i: (i, 0)),
            pl.BlockSpec((block_n, 1), lambda i: (i, 0)),
        ],
        out_shape=[
            jax.ShapeDtypeStruct((n, W_gl.shape[1]), jnp.float32),
            jax.ShapeDtypeStruct((n, W1.shape[1]), jnp.float32),
            jax.ShapeDtypeStruct((n, 1), jnp.float32),
        ],
    )(x, W_gl, W1, a)


def kernel(x, edge, num_nodes, W_gl, a, W1, W2):
    src = edge[0]
    dst = edge[1]
    n = x.shape[0]

    h, xw1, hn = _dense_stage(x, W_gl, W1, a)
    hn = hn[:, 0]
    U = jnp.max(hn)

    gs = h[src]
    gd = h[dst]
    s = jax.nn.relu(jnp.abs(gs - gd) @ a)[:, 0]
    ex = jnp.exp(s - hn[src] - U)

    rs = jax.ops.segment_sum(ex, src, num_segments=n)
    adj = ex / rs[src]

    acc1 = jax.ops.segment_sum(ex[:, None] * xw1[dst], src, num_segments=n)
    x1 = jax.nn.relu(jnp.where(rs[:, None] > 0, acc1 / rs[:, None], 0.0))

    xw2 = x1 @ W2
    output = jax.ops.segment_sum(adj[:, None] * xw2[dst], src, num_segments=n)
    return (output, adj, h)


# speedup vs baseline: 5.4059x; 4.4985x over previous
import functools
import jax
import jax.numpy as jnp
from jax import lax
from jax.experimental import pallas as pl
from jax.experimental.pallas import tpu as pltpu
from jax.experimental.pallas import tpu_sc as plsc

_SC_MESH = plsc.VectorSubcoreMesh(core_axis_name="c", subcore_axis_name="s")
_SC_PARAMS = pltpu.CompilerParams(use_tc_tiling_on_sc=False)


def _tc1_body(x_ref, wgl_ref, w1_ref, a_ref, h_ref, tdst_ref, u_ref):
    i = pl.program_id(0)
    x = x_ref[...]
    h = lax.dot(x, wgl_ref[...], preferred_element_type=jnp.float32)
    h_ref[...] = h
    tdst_ref[:, :64] = h
    tdst_ref[:, 64:] = lax.dot(x, w1_ref[...], preferred_element_type=jnp.float32)
    anorm = jnp.sqrt(jnp.sum(a_ref[...] ** 2))
    hn = anorm * jnp.sqrt(jnp.sum(h * h, axis=1, keepdims=True))
    bmax = jnp.max(hn).reshape(1, 1)
    u_ref[...] = jnp.where(i == 0, bmax, jnp.maximum(u_ref[...], bmax))


def _tc1(x, W_gl, W1, a, block_n=1000):
    n, d = x.shape
    return pl.pallas_call(
        _tc1_body,
        grid=(n // block_n,),
        in_specs=[
            pl.BlockSpec((block_n, d), lambda i: (i, 0)),
            pl.BlockSpec((d, 64), lambda i: (0, 0)),
            pl.BlockSpec((d, 32), lambda i: (0, 0)),
            pl.BlockSpec((64, 1), lambda i: (0, 0)),
        ],
        out_specs=[
            pl.BlockSpec((block_n, 64), lambda i: (i, 0)),
            pl.BlockSpec((block_n, 96), lambda i: (i, 0)),
            pl.BlockSpec((1, 1), lambda i: (0, 0)),
        ],
        out_shape=[
            jax.ShapeDtypeStruct((n, 64), jnp.float32),
            jax.ShapeDtypeStruct((n, 96), jnp.float32),
            jax.ShapeDtypeStruct((1, 1), jnp.float32),
        ],
    )(x, W_gl, W1, a)


def _sc_gather2(tab1, idx1, tab2, idx2, window):
    e = idx1.shape[1]
    d1 = tab1.shape[1]
    d2 = tab2.shape[1]

    @functools.partial(
        pl.kernel,
        out_type=(
            jax.ShapeDtypeStruct((e, d1), jnp.float32),
            jax.ShapeDtypeStruct((e, d2), jnp.float32),
        ),
        mesh=_SC_MESH,
        compiler_params=_SC_PARAMS,
    )
    def k(t1_hbm, i1_hbm, t2_hbm, i2_hbm, o1_hbm, o2_hbm):
        def body(i1_vmem, i2_vmem, o1_vmem, o2_vmem):
            pltpu.sync_copy(t1_hbm.at[i1_vmem.at[0]], o1_vmem)
            pltpu.sync_copy(t2_hbm.at[i2_vmem.at[0]], o2_vmem)

        pltpu.emit_pipeline(
            body,
            grid=(e // window,),
            in_specs=[
                pl.BlockSpec((1, window), lambda i: (0, i)),
                pl.BlockSpec((1, window), lambda i: (0, i)),
            ],
            out_specs=[
                pl.BlockSpec((window, d1), lambda i: (i, 0)),
                pl.BlockSpec((window, d2), lambda i: (i, 0)),
            ],
            core_axis_name=("c", "s"),
            dimension_semantics=(pltpu.PARALLEL,),
        )(i1_hbm, i2_hbm, o1_hbm, o2_hbm)

    return k(tab1, idx1, tab2, idx2)


def _tc2_body(gs_ref, gdx_ref, a_ref, u_ref, ex8_ref, m1_ref):
    gs = gs_ref[...]
    gd = gdx_ref[:, :64]
    a = a_ref[...]
    diff = jnp.abs(gs - gd)
    s = jax.nn.relu(lax.dot(diff, a, preferred_element_type=jnp.float32))
    anorm = jnp.sqrt(jnp.sum(a * a))
    u = anorm * jnp.sqrt(jnp.sum(gs * gs, axis=1, keepdims=True))
    ex = jnp.maximum(jnp.exp(s - u - u_ref[0, 0]), 1e-30)
    ex8_ref[...] = jnp.broadcast_to(ex, ex8_ref.shape)
    m1_ref[...] = ex * gdx_ref[:, 64:]


def _tc2(gs, gdx, a, U, block_e=4000):
    e = gs.shape[0]
    return pl.pallas_call(
        _tc2_body,
        grid=(e // block_e,),
        in_specs=[
            pl.BlockSpec((block_e, 64), lambda i: (i, 0)),
            pl.BlockSpec((block_e, 96), lambda i: (i, 0)),
            pl.BlockSpec((64, 1), lambda i: (0, 0)),
            pl.BlockSpec((1, 1), lambda i: (0, 0)),
        ],
        out_specs=[
            pl.BlockSpec((block_e, 8), lambda i: (i, 0)),
            pl.BlockSpec((block_e, 32), lambda i: (i, 0)),
        ],
        out_shape=[
            jax.ShapeDtypeStruct((e, 8), jnp.float32),
            jax.ShapeDtypeStruct((e, 32), jnp.float32),
        ],
    )(gs, gdx, a, U)


def _sc_scatter_add2(p1, p2, idx, n_seg, window):
    e, d1 = p1.shape
    d2 = p2.shape[1] if p2 is not None else 0

    out_types = [jax.ShapeDtypeStruct((2, n_seg, d1), jnp.float32)]
    scratches = [pltpu.VMEM_SHARED((n_seg, d1), jnp.float32)]
    if p2 is not None:
        out_types.append(jax.ShapeDtypeStruct((2, n_seg, d2), jnp.float32))
        scratches.append(pltpu.VMEM_SHARED((n_seg, d2), jnp.float32))

    @functools.partial(
        pl.kernel,
        out_type=tuple(out_types),
        mesh=_SC_MESH,
        scratch_types=scratches,
        compiler_params=_SC_PARAMS,
    )
    def k(*refs):
        if p2 is not None:
            p1_hbm, p2_hbm, idx_hbm, z1_hbm, z2_hbm, o1_hbm, o2_hbm, sh1, sh2 = refs
        else:
            p1_hbm, idx_hbm, z1_hbm, o1_hbm, sh1 = refs
        c = lax.axis_index("c")
        s = lax.axis_index("s")

        @pl.when(s == 0)
        def _():
            pltpu.sync_copy(z1_hbm, sh1)

        if p2 is not None:

            @pl.when(s == 1)
            def _():
                pltpu.sync_copy(z2_hbm, sh2)

        plsc.subcore_barrier()

        if p2 is not None:

            def body(p1_vmem, p2_vmem, idx_vmem):
                pltpu.sync_copy(p1_vmem, sh1.at[idx_vmem.at[0]], add=True)
                pltpu.sync_copy(p2_vmem, sh2.at[idx_vmem.at[0]], add=True)

            in_specs = [
                pl.BlockSpec((window, d1), lambda i: (i, 0)),
                pl.BlockSpec((window, d2), lambda i: (i, 0)),
                pl.BlockSpec((1, window), lambda i: (0, i)),
            ]
            args = (p1_hbm, p2_hbm, idx_hbm)
        else:

            def body(p1_vmem, idx_vmem):
                pltpu.sync_copy(p1_vmem, sh1.at[idx_vmem.at[0]], add=True)

            in_specs = [
                pl.BlockSpec((window, d1), lambda i: (i, 0)),
                pl.BlockSpec((1, window), lambda i: (0, i)),
            ]
            args = (p1_hbm, idx_hbm)

        pltpu.emit_pipeline(
            body,
            grid=(e // window,),
            in_specs=in_specs,
            out_specs=[],
            core_axis_name=("c", "s"),
            dimension_semantics=(pltpu.PARALLEL,),
        )(*args)

        plsc.subcore_barrier()

        @pl.when(s == 0)
        def _():
            pltpu.sync_copy(sh1, o1_hbm.at[c])

        if p2 is not None:

            @pl.when(s == 1)
            def _():
                pltpu.sync_copy(sh2, o2_hbm.at[c])

    z1 = jnp.zeros((n_seg, d1), jnp.float32)
    if p2 is not None:
        z2 = jnp.zeros((n_seg, d2), jnp.float32)
        return k(p1, p2, idx, z1, z2)
    return k(p1, idx, z1)


def _tc3_body(a32_ref, a8_ref, w2_ref, xw2_ref, rs8_ref):
    acc = a32_ref[0] + a32_ref[1]
    rs = a8_ref[0, :, 0:1] + a8_ref[1, :, 0:1]
    x1 = jax.nn.relu(jnp.where(rs > 0, acc / rs, 0.0))
    xw2_ref[...] = lax.dot(x1, w2_ref[...], preferred_element_type=jnp.float32)
    rs8_ref[...] = jnp.broadcast_to(rs, rs8_ref.shape)


def _tc3(acc32, acc8, W2, block_n=1000):
    n = acc32.shape[1]
    return pl.pallas_call(
        _tc3_body,
        grid=(n // block_n,),
        in_specs=[
            pl.BlockSpec((2, block_n, 32), lambda i: (0, i, 0)),
            pl.BlockSpec((2, block_n, 8), lambda i: (0, i, 0)),
            pl.BlockSpec((32, 16), lambda i: (0, 0)),
        ],
        out_specs=[
            pl.BlockSpec((block_n, 16), lambda i: (i, 0)),
            pl.BlockSpec((block_n, 8), lambda i: (i, 0)),
        ],
        out_shape=[
            jax.ShapeDtypeStruct((n, 16), jnp.float32),
            jax.ShapeDtypeStruct((n, 8), jnp.float32),
        ],
    )(acc32, acc8, W2)


def _tc4_body(g2_ref, rsg_ref, ex8_ref, adj_ref, m2_ref):
    ex = ex8_ref[:, 0:1]
    adj = ex / rsg_ref[:, 0:1]
    adj_ref[...] = adj
    m2_ref[...] = adj * g2_ref[...]


def _tc4(g2, rsg, ex8, block_e=4000):
    e = g2.shape[0]
    return pl.pallas_call(
        _tc4_body,
        grid=(e // block_e,),
        in_specs=[
            pl.BlockSpec((block_e, 16), lambda i: (i, 0)),
            pl.BlockSpec((block_e, 8), lambda i: (i, 0)),
            pl.BlockSpec((block_e, 8), lambda i: (i, 0)),
        ],
        out_specs=[
            pl.BlockSpec((block_e, 1), lambda i: (i, 0)),
            pl.BlockSpec((block_e, 16), lambda i: (i, 0)),
        ],
        out_shape=[
            jax.ShapeDtypeStruct((e, 1), jnp.float32),
            jax.ShapeDtypeStruct((e, 16), jnp.float32),
        ],
    )(g2, rsg, ex8)


def _tc5_body(p_ref, o_ref):
    o_ref[...] = p_ref[0] + p_ref[1]


def _tc5(parts, block_n=1000):
    n, d = parts.shape[1], parts.shape[2]
    return pl.pallas_call(
        _tc5_body,
        grid=(n // block_n,),
        in_specs=[pl.BlockSpec((2, block_n, d), lambda i: (0, i, 0))],
        out_specs=pl.BlockSpec((block_n, d), lambda i: (i, 0)),
        out_shape=jax.ShapeDtypeStruct((n, d), jnp.float32),
    )(parts)


def kernel(x, edge, num_nodes, W_gl, a, W1, W2):
    n = x.shape[0]
    e = edge.shape[1]
    src2 = edge[0:1]
    dst2 = edge[1:2]

    h, tdst, U = _tc1(x, W_gl, W1, a)
    gs, gdx = _sc_gather2(h, src2, tdst, dst2, window=200)
    ex8, m1 = _tc2(gs, gdx, a, U)
    acc32, acc8 = _sc_scatter_add2(m1, ex8, src2, n, window=400)
    xw2, rs8 = _tc3(acc32, acc8, W2)
    g2, rsg = _sc_gather2(xw2, dst2, rs8, src2, window=400)
    adj2, m2 = _tc4(g2, rsg, ex8)
    parts = _sc_scatter_add2(m2, None, src2, n, window=400)[0]
    output = _tc5(parts)
    return (output, adj2[:, 0], h)


# speedup vs baseline: 8.9050x; 1.6473x over previous
import functools
import jax
import jax.numpy as jnp
from jax import lax
from jax.experimental import pallas as pl
from jax.experimental.pallas import tpu as pltpu
from jax.experimental.pallas import tpu_sc as plsc

_MESH = plsc.VectorSubcoreMesh(core_axis_name="c", subcore_axis_name="s")
_PARAMS = pltpu.CompilerParams(use_tc_tiling_on_sc=False,
                               needs_layout_passes=False)
_W = 400


def _tc1_body(x_ref, wgl_ref, w1_ref, a_ref, h_ref, tdst_ref, hn8_ref, u_ref,
              abt_ref):
    i = pl.program_id(0)
    x = x_ref[...]
    h = lax.dot(x, wgl_ref[...], preferred_element_type=jnp.float32)
    h_ref[...] = h
    tdst_ref[:, :64] = h
    tdst_ref[:, 64:] = lax.dot(x, w1_ref[...], preferred_element_type=jnp.float32)
    anorm = jnp.sqrt(jnp.sum(a_ref[...] ** 2))
    hn = anorm * jnp.sqrt(jnp.sum(h * h, axis=1, keepdims=True))
    hn8_ref[...] = jnp.broadcast_to(hn, hn8_ref.shape)
    bmax = jnp.max(hn).reshape(1, 1)
    u_ref[...] = jnp.where(i == 0, bmax, jnp.maximum(u_ref[...], bmax))
    abt_ref[...] = jnp.broadcast_to(a_ref[...], abt_ref.shape)


def _tc1(x, W_gl, W1, a, block_n=1000):
    n, d = x.shape
    return pl.pallas_call(
        _tc1_body,
        grid=(n // block_n,),
        in_specs=[
            pl.BlockSpec((block_n, d), lambda i: (i, 0)),
            pl.BlockSpec((d, 64), lambda i: (0, 0)),
            pl.BlockSpec((d, 32), lambda i: (0, 0)),
            pl.BlockSpec((64, 1), lambda i: (0, 0)),
        ],
        out_specs=[
            pl.BlockSpec((block_n, 64), lambda i: (i, 0)),
            pl.BlockSpec((block_n, 96), lambda i: (i, 0)),
            pl.BlockSpec((block_n, 8), lambda i: (i, 0)),
            pl.BlockSpec((1, 1), lambda i: (0, 0)),
            pl.BlockSpec((64, 16), lambda i: (0, 0)),
        ],
        out_shape=[
            jax.ShapeDtypeStruct((n, 64), jnp.float32),
            jax.ShapeDtypeStruct((n, 96), jnp.float32),
            jax.ShapeDtypeStruct((n, 8), jnp.float32),
            jax.ShapeDtypeStruct((1, 1), jnp.float32),
            jax.ShapeDtypeStruct((64, 16), jnp.float32),
        ],
    )(x, W_gl, W1, a)


def _tc1b_body(h_ref, hn8_ref, u_ref, tsrc_ref):
    tsrc_ref[:, :64] = h_ref[...]
    tsrc_ref[:, 64:] = hn8_ref[...] + u_ref[0, 0]


def _tc1b(h, hn8, U, block_n=1000):
    n = h.shape[0]
    return pl.pallas_call(
        _tc1b_body,
        grid=(n // block_n,),
        in_specs=[
            pl.BlockSpec((block_n, 64), lambda i: (i, 0)),
            pl.BlockSpec((block_n, 8), lambda i: (i, 0)),
            pl.BlockSpec((1, 1), lambda i: (0, 0)),
        ],
        out_specs=pl.BlockSpec((block_n, 72), lambda i: (i, 0)),
        out_shape=jax.ShapeDtypeStruct((n, 72), jnp.float32),
    )(h, hn8, U)


def _sc1(tsrc, tdst, src2, dst2, abt, n, e):
    @functools.partial(
        pl.kernel,
        out_type=(
            jax.ShapeDtypeStruct((1, e), jnp.float32),
            jax.ShapeDtypeStruct((2, n, 40), jnp.float32),
        ),
        mesh=_MESH,
        scratch_types=[
            pltpu.VMEM((_W, 72), jnp.float32),
            pltpu.VMEM((_W, 96), jnp.float32),
            pltpu.VMEM((_W, 40), jnp.float32),
            pltpu.VMEM((64, 16), jnp.float32),
            pltpu.VMEM_SHARED((n, 40), jnp.float32),
        ],
        compiler_params=_PARAMS,
    )
    def k(tsrc_hbm, tdst_hbm, src_hbm, dst_hbm, abt_hbm, z_hbm,
          ex_hbm, acc_hbm, gs_v, gd_v, p_v, abt_v, sh):
        c = lax.axis_index("c")
        s = lax.axis_index("s")
        pltpu.sync_copy(abt_hbm, abt_v)

        @pl.when(s == 0)
        def _():
            pltpu.sync_copy(z_hbm, sh)

        plsc.subcore_barrier()

        rows0 = lax.iota(jnp.int32, 16)

        def body(src_v, dst_v, ex_v):
            pltpu.sync_copy(tsrc_hbm.at[src_v.at[0]], gs_v)
            pltpu.sync_copy(tdst_hbm.at[dst_v.at[0]], gd_v)

            @pl.loop(0, _W // 16)
            def _(g):
                rows = rows0 + g * 16
                acc = jnp.zeros((16,), jnp.float32)
                for kk in range(64):
                    ck = jnp.full((16,), kk, jnp.int32)
                    vs = plsc.load_gather(gs_v, [rows, ck])
                    vd = plsc.load_gather(gd_v, [rows, ck])
                    acc = acc + jnp.abs(vs - vd) * abt_v[kk, :]
                hnu = plsc.load_gather(gs_v, [rows, jnp.full((16,), 64, jnp.int32)])
                ex = jnp.maximum(jnp.exp(jnp.maximum(acc, 0.0) - hnu), 1e-30)
                ex_v[0, pl.ds(g * 16, 16)] = ex
                for kk in range(32):
                    col = plsc.load_gather(
                        gd_v, [rows, jnp.full((16,), 64 + kk, jnp.int32)])
                    plsc.store_scatter(
                        p_v, [rows, jnp.full((16,), kk, jnp.int32)], ex * col)
                plsc.store_scatter(
                    p_v, [rows, jnp.full((16,), 32, jnp.int32)], ex)

            pltpu.sync_copy(p_v, sh.at[src_v.at[0]], add=True)

        pltpu.emit_pipeline(
            body,
            grid=(e // _W,),
            in_specs=[
                pl.BlockSpec((1, _W), lambda i: (0, i)),
                pl.BlockSpec((1, _W), lambda i: (0, i)),
            ],
            out_specs=[pl.BlockSpec((1, _W), lambda i: (0, i))],
            core_axis_name=("c", "s"),
            dimension_semantics=(pltpu.PARALLEL,),
        )(src_hbm, dst_hbm, ex_hbm)

        plsc.subcore_barrier()

        @pl.when(s == 0)
        def _():
            pltpu.sync_copy(sh, acc_hbm.at[c])

    z = jnp.zeros((n, 40), jnp.float32)
    return k(tsrc, tdst, src2, dst2, abt, z)


def _tc3_body(acc_ref, w2_ref, xw2_ref, rs8_ref):
    tot = acc_ref[0] + acc_ref[1]
    rs = tot[:, 32:33]
    x1 = jax.nn.relu(jnp.where(rs > 0, tot[:, :32] / rs, 0.0))
    xw2_ref[...] = lax.dot(x1, w2_ref[...], preferred_element_type=jnp.float32)
    rs8_ref[...] = jnp.broadcast_to(rs, rs8_ref.shape)


def _tc3(acc, W2, block_n=1000):
    n = acc.shape[1]
    return pl.pallas_call(
        _tc3_body,
        grid=(n // block_n,),
        in_specs=[
            pl.BlockSpec((2, block_n, 40), lambda i: (0, i, 0)),
            pl.BlockSpec((32, 16), lambda i: (0, 0)),
        ],
        out_specs=[
            pl.BlockSpec((block_n, 16), lambda i: (i, 0)),
            pl.BlockSpec((block_n, 8), lambda i: (i, 0)),
        ],
        out_shape=[
            jax.ShapeDtypeStruct((n, 16), jnp.float32),
            jax.ShapeDtypeStruct((n, 8), jnp.float32),
        ],
    )(acc, W2)


def _sc2(xw2, rs8, ex, src2, dst2, n, e):
    @functools.partial(
        pl.kernel,
        out_type=(
            jax.ShapeDtypeStruct((1, e), jnp.float32),
            jax.ShapeDtypeStruct((2, n, 16), jnp.float32),
        ),
        mesh=_MESH,
        scratch_types=[
            pltpu.VMEM((_W, 16), jnp.float32),
            pltpu.VMEM((_W, 8), jnp.float32),
            pltpu.VMEM((_W, 16), jnp.float32),
            pltpu.VMEM_SHARED((n, 16), jnp.float32),
        ],
        compiler_params=_PARAMS,
    )
    def k(xw2_hbm, rs8_hbm, ex_hbm, src_hbm, dst_hbm, z_hbm,
          adj_hbm, out_hbm, g2_v, rs_v, p2_v, sh):
        c = lax.axis_index("c")
        s = lax.axis_index("s")

        @pl.when(s == 0)
        def _():
            pltpu.sync_copy(z_hbm, sh)

        plsc.subcore_barrier()

        rows0 = lax.iota(jnp.int32, 16)

        def body(ex_v, src_v, dst_v, adj_v):
            pltpu.sync_copy(xw2_hbm.at[dst_v.at[0]], g2_v)
            pltpu.sync_copy(rs8_hbm.at[src_v.at[0]], rs_v)

            @pl.loop(0, _W // 16)
            def _(g):
                rows = rows0 + g * 16
                exv = ex_v[0, pl.ds(g * 16, 16)]
                rsv = plsc.load_gather(rs_v, [rows, jnp.full((16,), 0, jnp.int32)])
                adj = exv / rsv
                adj_v[0, pl.ds(g * 16, 16)] = adj
                for kk in range(16):
                    col = plsc.load_gather(
                        g2_v, [rows, jnp.full((16,), kk, jnp.int32)])
                    plsc.store_scatter(
                        p2_v, [rows, jnp.full((16,), kk, jnp.int32)], adj * col)

            pltpu.sync_copy(p2_v, sh.at[src_v.at[0]], add=True)

        pltpu.emit_pipeline(
            body,
            grid=(e // _W,),
            in_specs=[
                pl.BlockSpec((1, _W), lambda i: (0, i)),
                pl.BlockSpec((1, _W), lambda i: (0, i)),
                pl.BlockSpec((1, _W), lambda i: (0, i)),
            ],
            out_specs=[pl.BlockSpec((1, _W), lambda i: (0, i))],
            core_axis_name=("c", "s"),
            dimension_semantics=(pltpu.PARALLEL,),
        )(ex_hbm, src_hbm, dst_hbm, adj_hbm)

        plsc.subcore_barrier()

        @pl.when(s == 0)
        def _():
            pltpu.sync_copy(sh, out_hbm.at[c])

    z = jnp.zeros((n, 16), jnp.float32)
    return k(xw2, rs8, ex, src2, dst2, z)


def _tc5_body(p_ref, o_ref):
    o_ref[...] = p_ref[0] + p_ref[1]


def _tc5(parts, block_n=1000):
    n, d = parts.shape[1], parts.shape[2]
    return pl.pallas_call(
        _tc5_body,
        grid=(n // block_n,),
        in_specs=[pl.BlockSpec((2, block_n, d), lambda i: (0, i, 0))],
        out_specs=pl.BlockSpec((block_n, d), lambda i: (i, 0)),
        out_shape=jax.ShapeDtypeStruct((n, d), jnp.float32),
    )(parts)


def kernel(x, edge, num_nodes, W_gl, a, W1, W2):
    n = x.shape[0]
    e = edge.shape[1]
    src2 = edge[0:1]
    dst2 = edge[1:2]

    h, tdst, hn8, U, abt = _tc1(x, W_gl, W1, a)
    tsrc = _tc1b(h, hn8, U)
    ex, acc = _sc1(tsrc, tdst, src2, dst2, abt, n, e)
    xw2, rs8 = _tc3(acc, W2)
    adj, parts = _sc2(xw2, rs8, ex, src2, dst2, n, e)
    output = _tc5(parts)
    return (output, adj[0], h)
